# Initial kernel scaffold; baseline (speedup 1.0000x reference)
import jax
import jax.numpy as jnp
from jax.experimental import pallas as pl


def kernel(x, gate_w, expert_w, expert_b, agg_w, agg_b, orig_w, orig_b):
    raise NotImplementedError("write your pallas kernel here")


# speedup vs baseline: 2.6726x; 2.6726x over previous
import functools

import jax
import jax.numpy as jnp
from jax.experimental import pallas as pl

E = 8
TS = 256


def _prep_kernel(gw_ref, ew_ref, eb_ref, aw_ref, ab_ref, wrows_ref, b16_ref):
    e = pl.program_id(0)
    aw = aw_ref[...]

    @pl.when(e == 0)
    def _():
        wrows_ref[0:E, :] = gw_ref[...]
        c = jax.lax.dot_general(
            aw, eb_ref[...], (((1,), (1,)), ((), ())),
            preferred_element_type=jnp.float32)
        b16_ref[...] = jnp.concatenate(
            [jnp.zeros((1, E), jnp.float32), c + ab_ref[...]], axis=1)

    v = jnp.dot(aw, ew_ref[0], preferred_element_type=jnp.float32)
    wrows_ref[pl.ds(E + e, 1), :] = v


def _main_kernel(x_ref, wcat_ref, b16_ref, owt_ref, ob_ref, out_ref):
    xt = x_ref[...]
    res = jnp.dot(xt, wcat_ref[...], preferred_element_type=jnp.float32)
    res = res + b16_ref[...]
    lane = jax.lax.broadcasted_iota(jnp.int32, res.shape, 1)
    neg = jnp.float32(-jnp.inf)
    lg = jnp.where(lane < E, res, neg)
    m1 = jnp.max(lg, axis=1, keepdims=True)
    a1 = jnp.min(jnp.where(lg == m1, lane, 2 * E), axis=1, keepdims=True)
    lg2 = jnp.where(lane == a1, neg, lg)
    m2 = jnp.max(lg2, axis=1, keepdims=True)
    a2 = jnp.min(jnp.where(lg2 == m2, lane, 2 * E), axis=1, keepdims=True)
    s1 = jnp.sum(jnp.where(lane == a1 + E, res, 0.0), axis=1, keepdims=True)
    s2 = jnp.sum(jnp.where(lane == a2 + E, res, 0.0), axis=1, keepdims=True)
    w1 = jax.nn.sigmoid(m1 - m2)
    agg = w1 * s1 + (1.0 - w1) * s2

    orig = jnp.dot(xt, owt_ref[...], preferred_element_type=jnp.float32)
    out_ref[...] = orig + ob_ref[...] + agg


@functools.partial(jax.jit, static_argnums=())
def kernel(x, gate_w, expert_w, expert_b, agg_w, agg_b, orig_w, orig_b):
    B, S, D = x.shape
    Ev, F, _ = expert_w.shape
    T = B * S

    wrows, b16 = pl.pallas_call(
        _prep_kernel,
        grid=(Ev,),
        in_specs=[
            pl.BlockSpec((Ev, D), lambda e: (0, 0)),
            pl.BlockSpec((1, F, D), lambda e: (e, 0, 0)),
            pl.BlockSpec((Ev, F), lambda e: (0, 0)),
            pl.BlockSpec((1, F), lambda e: (0, 0)),
            pl.BlockSpec((1, 1), lambda e: (0, 0)),
        ],
        out_specs=[
            pl.BlockSpec((2 * Ev, D), lambda e: (0, 0)),
            pl.BlockSpec((1, 2 * Ev), lambda e: (0, 0)),
        ],
        out_shape=[
            jax.ShapeDtypeStruct((2 * Ev, D), jnp.float32),
            jax.ShapeDtypeStruct((1, 2 * Ev), jnp.float32),
        ],
    )(gate_w, expert_w, expert_b, agg_w, agg_b.reshape(1, 1))

    out = pl.pallas_call(
        _main_kernel,
        grid=(T // TS,),
        in_specs=[
            pl.BlockSpec((TS, D), lambda i: (i, 0)),
            pl.BlockSpec((D, 2 * Ev), lambda i: (0, 0)),
            pl.BlockSpec((1, 2 * Ev), lambda i: (0, 0)),
            pl.BlockSpec((D, F), lambda i: (0, 0)),
            pl.BlockSpec((1, F), lambda i: (0, 0)),
        ],
        out_specs=pl.BlockSpec((TS, F), lambda i: (i, 0)),
        out_shape=jax.ShapeDtypeStruct((T, F), jnp.float32),
    )(x.reshape(T, D), wrows.T, b16, orig_w.T, orig_b.reshape(1, F))

    return out.reshape(B, S, F)


# speedup vs baseline: 2.7775x; 1.0392x over previous
import functools

import jax
import jax.numpy as jnp
from jax.experimental import pallas as pl

E = 8
TS = 256


def _prep_kernel(gw_ref, ew_ref, eb_ref, aw_ref, ab_ref, wrows_ref, b16_ref):
    e = pl.program_id(0)
    aw = aw_ref[...]

    @pl.when(e == 0)
    def _():
        wrows_ref[0:E, :] = gw_ref[...]
        c = jax.lax.dot_general(
            aw, eb_ref[...], (((1,), (1,)), ((), ())),
            preferred_element_type=jnp.float32)
        b16_ref[...] = jnp.concatenate(
            [jnp.zeros((1, E), jnp.float32), c + ab_ref[...]], axis=1)

    v = jnp.dot(aw, ew_ref[0], preferred_element_type=jnp.float32)
    wrows_ref[pl.ds(E + e, 1), :] = v


def _main_kernel(x_ref, wcat_ref, b16_ref, owt_ref, ob_ref, out_ref):
    xt = x_ref[...]
    res = jnp.dot(xt, wcat_ref[...], preferred_element_type=jnp.float32)
    res = res + b16_ref[...]
    lane = jax.lax.broadcasted_iota(jnp.int32, res.shape, 1)
    neg = jnp.float32(-jnp.inf)
    lg = jnp.where(lane < E, res, neg)
    m1 = jnp.max(lg, axis=1, keepdims=True)
    a1 = jnp.min(jnp.where(lg == m1, lane, 2 * E), axis=1, keepdims=True)
    lg2 = jnp.where(lane == a1, neg, lg)
    m2 = jnp.max(lg2, axis=1, keepdims=True)
    a2 = jnp.min(jnp.where(lg2 == m2, lane, 2 * E), axis=1, keepdims=True)
    s1 = jnp.sum(jnp.where(lane == a1 + E, res, 0.0), axis=1, keepdims=True)
    s2 = jnp.sum(jnp.where(lane == a2 + E, res, 0.0), axis=1, keepdims=True)
    w1 = jax.nn.sigmoid(m1 - m2)
    agg = w1 * s1 + (1.0 - w1) * s2

    orig = jnp.dot(xt.astype(jnp.bfloat16), owt_ref[...],
                   preferred_element_type=jnp.float32)
    out_ref[...] = orig + ob_ref[...] + agg


@functools.partial(jax.jit, static_argnums=())
def kernel(x, gate_w, expert_w, expert_b, agg_w, agg_b, orig_w, orig_b):
    B, S, D = x.shape
    Ev, F, _ = expert_w.shape
    T = B * S

    wrows, b16 = pl.pallas_call(
        _prep_kernel,
        grid=(Ev,),
        in_specs=[
            pl.BlockSpec((Ev, D), lambda e: (0, 0)),
            pl.BlockSpec((1, F, D), lambda e: (e, 0, 0)),
            pl.BlockSpec((Ev, F), lambda e: (0, 0)),
            pl.BlockSpec((1, F), lambda e: (0, 0)),
            pl.BlockSpec((1, 1), lambda e: (0, 0)),
        ],
        out_specs=[
            pl.BlockSpec((2 * Ev, D), lambda e: (0, 0)),
            pl.BlockSpec((1, 2 * Ev), lambda e: (0, 0)),
        ],
        out_shape=[
            jax.ShapeDtypeStruct((2 * Ev, D), jnp.float32),
            jax.ShapeDtypeStruct((1, 2 * Ev), jnp.float32),
        ],
    )(gate_w, expert_w, expert_b, agg_w, agg_b.reshape(1, 1))

    out = pl.pallas_call(
        _main_kernel,
        grid=(T // TS,),
        in_specs=[
            pl.BlockSpec((TS, D), lambda i: (i, 0)),
            pl.BlockSpec((D, 2 * Ev), lambda i: (0, 0)),
            pl.BlockSpec((1, 2 * Ev), lambda i: (0, 0)),
            pl.BlockSpec((D, F), lambda i: (0, 0)),
            pl.BlockSpec((1, F), lambda i: (0, 0)),
        ],
        out_specs=pl.BlockSpec((TS, F), lambda i: (i, 0)),
        out_shape=jax.ShapeDtypeStruct((T, F), jnp.float32),
    )(x.reshape(T, D), wrows.T, b16,
      orig_w.T.astype(jnp.bfloat16), orig_b.reshape(1, F))

    return out.reshape(B, S, F)


# speedup vs baseline: 2.9290x; 1.0545x over previous
import jax
import jax.numpy as jnp
from jax.experimental import pallas as pl

E = 8
TS = 256


def _fused_kernel(x_ref, ew_ref, gw_ref, eb_ref, aw_ref, ab_ref, owt_ref,
                  ob_ref, out_ref, wrows_s, b16_s):
    s = pl.program_id(0)
    aw = aw_ref[...]

    @pl.when(s == 0)
    def _():
        wrows_s[0:E, :] = gw_ref[...]
        c = jax.lax.dot_general(
            aw, eb_ref[...], (((1,), (1,)), ((), ())),
            preferred_element_type=jnp.float32)
        b16_s[...] = jnp.concatenate(
            [jnp.zeros((1, E), jnp.float32), c + ab_ref[...]], axis=1)

    @pl.when(s < E)
    def _():
        v = jnp.dot(aw, ew_ref[0], preferred_element_type=jnp.float32)
        wrows_s[pl.ds(E + s, 1), :] = v

    @pl.when(s >= E)
    def _():
        xt = x_ref[...]
        res = jax.lax.dot_general(
            xt, wrows_s[...], (((1,), (1,)), ((), ())),
            preferred_element_type=jnp.float32)
        res = res + b16_s[...]
        lane = jax.lax.broadcasted_iota(jnp.int32, res.shape, 1)
        neg = jnp.float32(-jnp.inf)
        lg = jnp.where(lane < E, res, neg)
        m1 = jnp.max(lg, axis=1, keepdims=True)
        a1 = jnp.min(jnp.where(lg == m1, lane, 2 * E), axis=1, keepdims=True)
        lg2 = jnp.where(lane == a1, neg, lg)
        m2 = jnp.max(lg2, axis=1, keepdims=True)
        a2 = jnp.min(jnp.where(lg2 == m2, lane, 2 * E), axis=1, keepdims=True)
        s1 = jnp.sum(jnp.where(lane == a1 + E, res, 0.0), axis=1, keepdims=True)
        s2 = jnp.sum(jnp.where(lane == a2 + E, res, 0.0), axis=1, keepdims=True)
        w1 = jax.nn.sigmoid(m1 - m2)
        agg = w1 * s1 + (1.0 - w1) * s2

        orig = jnp.dot(xt.astype(jnp.bfloat16), owt_ref[...],
                       preferred_element_type=jnp.float32)
        out_ref[...] = orig + ob_ref[...] + agg


def kernel(x, gate_w, expert_w, expert_b, agg_w, agg_b, orig_w, orig_b):
    B, S, D = x.shape
    Ev, F, _ = expert_w.shape
    T = B * S
    from jax.experimental.pallas import tpu as pltpu

    out = pl.pallas_call(
        _fused_kernel,
        grid=(Ev + T // TS,),
        in_specs=[
            pl.BlockSpec((TS, D), lambda s: (jnp.maximum(s - E, 0), 0)),
            pl.BlockSpec((1, F, D), lambda s: (jnp.minimum(s, E - 1), 0, 0)),
            pl.BlockSpec((Ev, D), lambda s: (0, 0)),
            pl.BlockSpec((Ev, F), lambda s: (0, 0)),
            pl.BlockSpec((1, F), lambda s: (0, 0)),
            pl.BlockSpec((1, 1), lambda s: (0, 0)),
            pl.BlockSpec((D, F), lambda s: (0, 0)),
            pl.BlockSpec((1, F), lambda s: (0, 0)),
        ],
        out_specs=pl.BlockSpec((TS, F), lambda s: (jnp.maximum(s - E, 0), 0)),
        out_shape=jax.ShapeDtypeStruct((T, F), jnp.float32),
        scratch_shapes=[
            pltpu.VMEM((2 * E, D), jnp.float32),
            pltpu.VMEM((1, 2 * E), jnp.float32),
        ],
    )(x.reshape(T, D), expert_w, gate_w, expert_b, agg_w,
      agg_b.reshape(1, 1), orig_w.T.astype(jnp.bfloat16), orig_b.reshape(1, F))

    return out.reshape(B, S, F)


# speedup vs baseline: 3.2651x; 1.1147x over previous
import jax
import jax.numpy as jnp
from jax.experimental import pallas as pl
from jax.experimental.pallas import tpu as pltpu

E = 8
TS = 512


def _fused_kernel(x_ref, ew_ref, gw_ref, eb_ref, aw_ref, ab_ref, owt_ref,
                  ob_ref, out_ref, wrows_s, b16_s):
    s = pl.program_id(0)
    aw = aw_ref[...]

    @pl.when(s == 0)
    def _():
        wrows_s[0:E, :] = gw_ref[...]
        c = jax.lax.dot_general(
            aw, eb_ref[...], (((1,), (1,)), ((), ())),
            preferred_element_type=jnp.float32)
        b16_s[...] = jnp.concatenate(
            [jnp.zeros((1, E), jnp.float32), c + ab_ref[...]], axis=1)

    @pl.when(s < E)
    def _():
        v = jnp.dot(aw, ew_ref[0], preferred_element_type=jnp.float32)
        wrows_s[pl.ds(E + s, 1), :] = v

    @pl.when(s >= E)
    def _():
        xt = x_ref[...]
        res = jax.lax.dot_general(
            xt, wrows_s[...], (((1,), (1,)), ((), ())),
            preferred_element_type=jnp.float32)
        res = res + b16_s[...]
        lg = res[:, 0:E]
        sc = res[:, E:2 * E]
        m1 = jnp.max(lg, axis=1, keepdims=True)
        neg = jnp.float32(-jnp.inf)
        m2 = jnp.max(jnp.where(lg < m1, lg, neg), axis=1, keepdims=True)
        p = jnp.exp(lg - m1)
        num = jnp.sum(jnp.where(lg >= m2, p * sc, 0.0), axis=1, keepdims=True)
        den = 1.0 + jnp.exp(m2 - m1)
        agg = num / den

        orig = jnp.dot(xt.astype(jnp.bfloat16), owt_ref[...],
                       preferred_element_type=jnp.float32)
        out_ref[...] = orig + ob_ref[...] + agg


def kernel(x, gate_w, expert_w, expert_b, agg_w, agg_b, orig_w, orig_b):
    B, S, D = x.shape
    Ev, F, _ = expert_w.shape
    T = B * S

    out = pl.pallas_call(
        _fused_kernel,
        grid=(Ev + T // TS,),
        in_specs=[
            pl.BlockSpec((TS, D), lambda s: (jnp.maximum(s - E, 0), 0)),
            pl.BlockSpec((1, F, D), lambda s: (jnp.minimum(s, E - 1), 0, 0)),
            pl.BlockSpec((Ev, D), lambda s: (0, 0)),
            pl.BlockSpec((Ev, F), lambda s: (0, 0)),
            pl.BlockSpec((1, F), lambda s: (0, 0)),
            pl.BlockSpec((1, 1), lambda s: (0, 0)),
            pl.BlockSpec((D, F), lambda s: (0, 0)),
            pl.BlockSpec((1, F), lambda s: (0, 0)),
        ],
        out_specs=pl.BlockSpec((TS, F), lambda s: (jnp.maximum(s - E, 0), 0)),
        out_shape=jax.ShapeDtypeStruct((T, F), jnp.float32),
        scratch_shapes=[
            pltpu.VMEM((2 * E, D), jnp.float32),
            pltpu.VMEM((1, 2 * E), jnp.float32),
        ],
    )(x.reshape(T, D), expert_w, gate_w, expert_b, agg_w,
      agg_b.reshape(1, 1), orig_w.T.astype(jnp.bfloat16), orig_b.reshape(1, F))

    return out.reshape(B, S, F)


# speedup vs baseline: 3.6306x; 1.1119x over previous
import jax
import jax.numpy as jnp
from jax.experimental import pallas as pl
from jax.experimental.pallas import tpu as pltpu

E = 8
TS = 512


def _fused_kernel(x_ref, ew_ref, gw_ref, eb_ref, aw_ref, ab_ref, ow_ref,
                  ob_ref, out_ref, wrows_s, b16_s, owt_s):
    s = pl.program_id(0)
    aw = aw_ref[...]

    @pl.when(s == 0)
    def _():
        wrows_s[0:E, :] = gw_ref[...]
        c = jax.lax.dot_general(
            aw, eb_ref[...], (((1,), (1,)), ((), ())),
            preferred_element_type=jnp.float32)
        b16_s[...] = jnp.concatenate(
            [jnp.zeros((1, E), jnp.float32), c + ab_ref[...]], axis=1)
        owt_s[...] = jnp.transpose(ow_ref[...]).astype(jnp.bfloat16)

    @pl.when(s < E)
    def _():
        v = jnp.dot(aw, ew_ref[0], preferred_element_type=jnp.float32)
        wrows_s[pl.ds(E + s, 1), :] = v

    @pl.when(s >= E)
    def _():
        xt = x_ref[...]
        res = jax.lax.dot_general(
            xt, wrows_s[...], (((1,), (1,)), ((), ())),
            preferred_element_type=jnp.float32)
        res = res + b16_s[...]
        lg = res[:, 0:E]
        sc = res[:, E:2 * E]
        m1 = jnp.max(lg, axis=1, keepdims=True)
        neg = jnp.float32(-jnp.inf)
        m2 = jnp.max(jnp.where(lg < m1, lg, neg), axis=1, keepdims=True)
        p = jnp.exp(lg - m1)
        num = jnp.sum(jnp.where(lg >= m2, p * sc, 0.0), axis=1, keepdims=True)
        den = 1.0 + jnp.exp(m2 - m1)
        agg = num / den

        orig = jnp.dot(xt.astype(jnp.bfloat16), owt_s[...],
                       preferred_element_type=jnp.float32)
        out_ref[...] = orig + ob_ref[...] + agg


def kernel(x, gate_w, expert_w, expert_b, agg_w, agg_b, orig_w, orig_b):
    B, S, D = x.shape
    Ev, F, _ = expert_w.shape
    T = B * S

    out = pl.pallas_call(
        _fused_kernel,
        grid=(Ev + T // TS,),
        in_specs=[
            pl.BlockSpec((TS, D), lambda s: (jnp.maximum(s - E, 0), 0)),
            pl.BlockSpec((1, F, D), lambda s: (jnp.minimum(s, E - 1), 0, 0)),
            pl.BlockSpec((Ev, D), lambda s: (0, 0)),
            pl.BlockSpec((Ev, F), lambda s: (0, 0)),
            pl.BlockSpec((1, F), lambda s: (0, 0)),
            pl.BlockSpec((1, 1), lambda s: (0, 0)),
            pl.BlockSpec((F, D), lambda s: (0, 0)),
            pl.BlockSpec((1, F), lambda s: (0, 0)),
        ],
        out_specs=pl.BlockSpec((TS, F), lambda s: (jnp.maximum(s - E, 0), 0)),
        out_shape=jax.ShapeDtypeStruct((T, F), jnp.float32),
        scratch_shapes=[
            pltpu.VMEM((2 * E, D), jnp.float32),
            pltpu.VMEM((1, 2 * E), jnp.float32),
            pltpu.VMEM((D, F), jnp.bfloat16),
        ],
    )(x.reshape(T, D), expert_w, gate_w, expert_b, agg_w,
      agg_b.reshape(1, 1), orig_w, orig_b.reshape(1, F))

    return out.reshape(B, S, F)


# speedup vs baseline: 3.7507x; 1.0331x over previous
import jax
import jax.numpy as jnp
from jax.experimental import pallas as pl
from jax.experimental.pallas import tpu as pltpu

E = 8
TS = 1024


def _fused_kernel(x_ref, ew_ref, gw_ref, eb_ref, aw_ref, ab_ref, ow_ref,
                  ob_ref, out_ref, wrows_s, b16_s, owt_s):
    s = pl.program_id(0)
    aw = aw_ref[...]

    @pl.when(s == 0)
    def _():
        wrows_s[0:E, :] = gw_ref[...]
        c = jax.lax.dot_general(
            aw, eb_ref[...], (((1,), (1,)), ((), ())),
            preferred_element_type=jnp.float32)
        b16_s[...] = jnp.concatenate(
            [jnp.zeros((1, E), jnp.float32), c + ab_ref[...]], axis=1)
        owt_s[...] = jnp.transpose(ow_ref[...]).astype(jnp.bfloat16)

    @pl.when(s < E)
    def _():
        v = jnp.dot(aw, ew_ref[0], preferred_element_type=jnp.float32)
        wrows_s[pl.ds(E + s, 1), :] = v

    @pl.when(s >= E)
    def _():
        xt = x_ref[...]
        res = jax.lax.dot_general(
            xt, wrows_s[...], (((1,), (1,)), ((), ())),
            preferred_element_type=jnp.float32)
        res = res + b16_s[...]
        lg = res[:, 0:E]
        sc = res[:, E:2 * E]
        m1 = jnp.max(lg, axis=1, keepdims=True)
        neg = jnp.float32(-jnp.inf)
        m2 = jnp.max(jnp.where(lg < m1, lg, neg), axis=1, keepdims=True)
        p = jnp.exp(lg - m1)
        num = jnp.sum(jnp.where(lg >= m2, p * sc, 0.0), axis=1, keepdims=True)
        den = 1.0 + jnp.exp(m2 - m1)
        agg = num / den

        orig = jnp.dot(xt.astype(jnp.bfloat16), owt_s[...],
                       preferred_element_type=jnp.float32)
        out_ref[...] = orig + ob_ref[...] + agg


def kernel(x, gate_w, expert_w, expert_b, agg_w, agg_b, orig_w, orig_b):
    B, S, D = x.shape
    Ev, F, _ = expert_w.shape
    T = B * S

    out = pl.pallas_call(
        _fused_kernel,
        grid=(Ev + T // TS,),
        in_specs=[
            pl.BlockSpec((TS, D), lambda s: (jnp.maximum(s - E, 0), 0)),
            pl.BlockSpec((1, F, D), lambda s: (jnp.minimum(s, E - 1), 0, 0)),
            pl.BlockSpec((Ev, D), lambda s: (0, 0)),
            pl.BlockSpec((Ev, F), lambda s: (0, 0)),
            pl.BlockSpec((1, F), lambda s: (0, 0)),
            pl.BlockSpec((1, 1), lambda s: (0, 0)),
            pl.BlockSpec((F, D), lambda s: (0, 0)),
            pl.BlockSpec((1, F), lambda s: (0, 0)),
        ],
        out_specs=pl.BlockSpec((TS, F), lambda s: (jnp.maximum(s - E, 0), 0)),
        out_shape=jax.ShapeDtypeStruct((T, F), jnp.float32),
        scratch_shapes=[
            pltpu.VMEM((2 * E, D), jnp.float32),
            pltpu.VMEM((1, 2 * E), jnp.float32),
            pltpu.VMEM((D, F), jnp.bfloat16),
        ],
    )(x.reshape(T, D), expert_w, gate_w, expert_b, agg_w,
      agg_b.reshape(1, 1), orig_w, orig_b.reshape(1, F))

    return out.reshape(B, S, F)


# speedup vs baseline: 3.8941x; 1.0382x over previous
import jax
import jax.numpy as jnp
from jax.experimental import pallas as pl
from jax.experimental.pallas import tpu as pltpu

E = 8
EC = 2
VS = E // EC
TS = 1024


def _fused_kernel(x_ref, ew_ref, gw_ref, eb_ref, aw_ref, ab_ref, ow_ref,
                  ob_ref, out_ref, wrows_s, b16_s, owt_s):
    s = pl.program_id(0)
    aw = aw_ref[...]

    @pl.when(s == 0)
    def _():
        wrows_s[0:E, :] = gw_ref[...]
        c = jax.lax.dot_general(
            aw, eb_ref[...], (((1,), (1,)), ((), ())),
            preferred_element_type=jnp.float32)
        b16_s[...] = jnp.concatenate(
            [jnp.zeros((1, E), jnp.float32), c + ab_ref[...]], axis=1)
        owt_s[...] = jnp.transpose(ow_ref[...]).astype(jnp.bfloat16)

    @pl.when(s < VS)
    def _():
        for j in range(EC):
            v = jnp.dot(aw, ew_ref[j], preferred_element_type=jnp.float32)
            wrows_s[pl.ds(E + s * EC + j, 1), :] = v

    @pl.when(s >= VS)
    def _():
        xt = x_ref[...]
        res = jax.lax.dot_general(
            xt, wrows_s[...], (((1,), (1,)), ((), ())),
            preferred_element_type=jnp.float32)
        res = res + b16_s[...]
        lg = res[:, 0:E]
        sc = res[:, E:2 * E]
        m1 = jnp.max(lg, axis=1, keepdims=True)
        neg = jnp.float32(-jnp.inf)
        m2 = jnp.max(jnp.where(lg < m1, lg, neg), axis=1, keepdims=True)
        p = jnp.exp(lg - m1)
        num = jnp.sum(jnp.where(lg >= m2, p * sc, 0.0), axis=1, keepdims=True)
        den = 1.0 + jnp.exp(m2 - m1)
        agg = num / den

        orig = jnp.dot(xt.astype(jnp.bfloat16), owt_s[...],
                       preferred_element_type=jnp.float32)
        out_ref[...] = orig + ob_ref[...] + agg


def kernel(x, gate_w, expert_w, expert_b, agg_w, agg_b, orig_w, orig_b):
    B, S, D = x.shape
    Ev, F, _ = expert_w.shape
    T = B * S

    out = pl.pallas_call(
        _fused_kernel,
        grid=(VS + T // TS,),
        in_specs=[
            pl.BlockSpec((TS, D), lambda s: (jnp.maximum(s - VS, 0), 0)),
            pl.BlockSpec((EC, F, D), lambda s: (jnp.minimum(s, VS - 1), 0, 0)),
            pl.BlockSpec((Ev, D), lambda s: (0, 0)),
            pl.BlockSpec((Ev, F), lambda s: (0, 0)),
            pl.BlockSpec((1, F), lambda s: (0, 0)),
            pl.BlockSpec((1, 1), lambda s: (0, 0)),
            pl.BlockSpec((F, D), lambda s: (0, 0)),
            pl.BlockSpec((1, F), lambda s: (0, 0)),
        ],
        out_specs=pl.BlockSpec((TS, F), lambda s: (jnp.maximum(s - VS, 0), 0)),
        out_shape=jax.ShapeDtypeStruct((T, F), jnp.float32),
        scratch_shapes=[
            pltpu.VMEM((2 * E, D), jnp.float32),
            pltpu.VMEM((1, 2 * E), jnp.float32),
            pltpu.VMEM((D, F), jnp.bfloat16),
        ],
    )(x.reshape(T, D), expert_w, gate_w, expert_b, agg_w,
      agg_b.reshape(1, 1), orig_w, orig_b.reshape(1, F))

    return out.reshape(B, S, F)
